# baseline (speedup 1.0000x reference)
import functools

import jax
import jax.numpy as jnp
from jax import lax
from jax.experimental import pallas as pl
from jax.experimental.pallas import tpu as pltpu
from jax.experimental.pallas import tpu_sc as plsc

DIM = 768
NUM_EXPERTS = 8
TOP_K = 2
HID = int(DIM * 1.5)
N_TOKENS = 2048
N_PAIRS = N_TOKENS * TOP_K
LANES = 128
BT = 256
S_MAX = N_PAIRS + NUM_EXPERTS * BT
T_MAX = S_MAX // BT
NW = 32
GCH = 64


def _router_body(x_ref, rw_ref, ii_ref, iw_ref):
    x = x_ref[...]
    rw = rw_ref[...]
    logits = lax.dot_general(
        x, rw, (((1,), (1,)), ((), ())), preferred_element_type=jnp.float32)
    lane = lax.broadcasted_iota(jnp.int32, logits.shape, 1)
    neg_inf = jnp.float32(-jnp.inf)
    logits = jnp.where(lane < NUM_EXPERTS, logits, neg_inf)
    m1 = jnp.max(logits, axis=1, keepdims=True)
    a1 = jnp.min(jnp.where(logits == m1, lane, LANES), axis=1, keepdims=True)
    masked = jnp.where(lane == a1, neg_inf, logits)
    m2 = jnp.max(masked, axis=1, keepdims=True)
    a2 = jnp.min(jnp.where(masked == m2, lane, LANES), axis=1, keepdims=True)
    p1 = jax.nn.sigmoid(m1 - m2)
    p2 = 1.0 - p1
    ii_ref[...] = jnp.where(lane == 0, a1, jnp.where(lane == 1, a2, 0))
    iw_ref[...] = jnp.where(lane == 0, p1, jnp.where(lane == 1, p2, 0.0))


def _sc_gather_body(x_hbm, idx_hbm, out_hbm, idxc_v, buf_v, sem):
    wid = lax.axis_index("s") * 2 + lax.axis_index("c")
    rows_per_w = S_MAX // NW
    base = wid * rows_per_w
    for j in range(rows_per_w // GCH):
        pltpu.sync_copy(idx_hbm.at[pl.ds(base + j * GCH, GCH)], idxc_v)
        pltpu.async_copy(x_hbm.at[idxc_v], buf_v, sem).wait()
        pltpu.sync_copy(buf_v, out_hbm.at[pl.ds(base + j * GCH, GCH)])


def _dispatch_body(te_ref, act_ref, xs_ref, w1_ref, w2_ref, w3_ref, rw_ref,
                   out_ref):
    t = pl.program_id(0)

    @pl.when(act_ref[t] == 1)
    def _():
        x = xs_ref[...]
        w1 = w1_ref[0]
        w3 = w3_ref[0]
        w2 = w2_ref[0]
        g = lax.dot_general(
            x, w1, (((1,), (1,)), ((), ())), preferred_element_type=jnp.float32)
        u = lax.dot_general(
            x, w3, (((1,), (1,)), ((), ())), preferred_element_type=jnp.float32)
        h = g * jax.nn.sigmoid(g) * u
        y = lax.dot_general(
            h, w2, (((1,), (1,)), ((), ())), preferred_element_type=jnp.float32)
        out_ref[...] = y * rw_ref[:, 0:1]


def _sc_combine_body(ys_hbm, pos0_hbm, pos1_hbm, out_hbm, i0_v, i1_v,
                     buf0_v, buf1_v, sem):
    wid = lax.axis_index("s") * 2 + lax.axis_index("c")
    rows_per_w = N_TOKENS // NW
    base = wid * rows_per_w
    pltpu.sync_copy(pos0_hbm.at[pl.ds(base, rows_per_w)], i0_v)
    pltpu.sync_copy(pos1_hbm.at[pl.ds(base, rows_per_w)], i1_v)
    c0 = pltpu.async_copy(ys_hbm.at[i0_v], buf0_v, sem)
    c1 = pltpu.async_copy(ys_hbm.at[i1_v], buf1_v, sem)
    c0.wait()
    c1.wait()

    def row(r, carry):
        for c in range(DIM // 16):
            sl = pl.ds(c * 16, 16)
            buf0_v[r, sl] = buf0_v[r, sl] + buf1_v[r, sl]
        return carry

    lax.fori_loop(0, rows_per_w, row, 0)
    pltpu.sync_copy(buf0_v, out_hbm.at[pl.ds(base, rows_per_w)])


def kernel(x, router_w, w1, w2, w3):
    B, T, C = x.shape
    x_flat = x.reshape(-1, C)
    n = x_flat.shape[0]
    rw_pad = jnp.zeros((LANES, C), x.dtype).at[:NUM_EXPERTS].set(router_w)

    ii, iw = pl.pallas_call(
        _router_body,
        out_shape=(
            jax.ShapeDtypeStruct((n, LANES), jnp.int32),
            jax.ShapeDtypeStruct((n, LANES), jnp.float32),
        ),
        in_specs=[
            pl.BlockSpec((n, C), lambda: (0, 0)),
            pl.BlockSpec((LANES, C), lambda: (0, 0)),
        ],
        out_specs=(
            pl.BlockSpec((n, LANES), lambda: (0, 0)),
            pl.BlockSpec((n, LANES), lambda: (0, 0)),
        ),
    )(x_flat, rw_pad)

    e_pair = ii[:, :TOP_K].reshape(-1)
    w_pair = iw[:, :TOP_K].reshape(-1)
    tok = jnp.arange(N_PAIRS, dtype=jnp.int32) // TOP_K
    onehot = (e_pair[:, None] == jnp.arange(NUM_EXPERTS)[None, :])
    onehot = onehot.astype(jnp.int32)
    csum = jnp.cumsum(onehot, axis=0)
    counts = csum[-1]
    rank = jnp.take_along_axis(csum, e_pair[:, None], axis=1)[:, 0] - 1
    padded = ((counts + BT - 1) // BT) * BT
    pstart = jnp.cumsum(padded) - padded
    pos_pair = (pstart[e_pair] + rank).astype(jnp.int32)
    src_tok = jnp.zeros((S_MAX,), jnp.int32).at[pos_pair].set(tok)
    row_w = jnp.zeros((S_MAX,), jnp.float32).at[pos_pair].set(w_pair)
    row_w_b = jnp.broadcast_to(row_w[:, None], (S_MAX, 128))
    pos0 = pos_pair[0::TOP_K]
    pos1 = pos_pair[1::TOP_K]
    bound = jnp.cumsum(padded // BT)
    t_ar = jnp.arange(T_MAX, dtype=jnp.int32)
    tile_e = jnp.sum((t_ar[:, None] >= bound[None, :]).astype(jnp.int32), axis=1)
    active = (tile_e < NUM_EXPERTS).astype(jnp.int32)
    tile_e = jnp.minimum(tile_e, NUM_EXPERTS - 1)

    mesh = plsc.VectorSubcoreMesh(
        core_axis_name="c", subcore_axis_name="s", num_cores=2, num_subcores=16)
    xs = pl.kernel(
        _sc_gather_body,
        out_type=jax.ShapeDtypeStruct((S_MAX, C), jnp.float32),
        mesh=mesh,
        scratch_types=[
            pltpu.VMEM((GCH,), jnp.int32),
            pltpu.VMEM((GCH, C), jnp.float32),
            pltpu.SemaphoreType.DMA,
        ],
    )(x_flat, src_tok)

    grid_spec = pltpu.PrefetchScalarGridSpec(
        num_scalar_prefetch=2,
        grid=(T_MAX,),
        in_specs=[
            pl.BlockSpec((BT, C), lambda t, te, act: (t, 0)),
            pl.BlockSpec((1, HID, C), lambda t, te, act: (te[t], 0, 0)),
            pl.BlockSpec((1, C, HID), lambda t, te, act: (te[t], 0, 0)),
            pl.BlockSpec((1, HID, C), lambda t, te, act: (te[t], 0, 0)),
            pl.BlockSpec((BT, 128), lambda t, te, act: (t, 0)),
        ],
        out_specs=pl.BlockSpec((BT, C), lambda t, te, act: (t, 0)),
    )
    ysw = pl.pallas_call(
        _dispatch_body,
        grid_spec=grid_spec,
        out_shape=jax.ShapeDtypeStruct((S_MAX, C), jnp.float32),
    )(tile_e, active, xs, w1, w2, w3, row_w_b)

    out = pl.kernel(
        _sc_combine_body,
        out_type=jax.ShapeDtypeStruct((n, C), jnp.float32),
        mesh=mesh,
        scratch_types=[
            pltpu.VMEM((N_TOKENS // NW,), jnp.int32),
            pltpu.VMEM((N_TOKENS // NW,), jnp.int32),
            pltpu.VMEM((N_TOKENS // NW, C), jnp.float32),
            pltpu.VMEM((N_TOKENS // NW, C), jnp.float32),
            pltpu.SemaphoreType.DMA,
        ],
    )(ysw, pos0, pos1)
    return out.reshape(B, T, C)


# speedup vs baseline: 1.5760x; 1.5760x over previous
import functools

import jax
import jax.numpy as jnp
from jax import lax
from jax.experimental import pallas as pl
from jax.experimental.pallas import tpu as pltpu
from jax.experimental.pallas import tpu_sc as plsc

DIM = 768
NUM_EXPERTS = 8
TOP_K = 2
HID = int(DIM * 1.5)
N_TOKENS = 2048
N_PAIRS = N_TOKENS * TOP_K
LANES = 128
BT = 256
S_MAX = N_PAIRS + NUM_EXPERTS * BT
T_MAX = S_MAX // BT
NW = 32
GCH = 64


def _router_body(x_ref, rw_ref, ii_ref, iw_ref):
    x = x_ref[...]
    rw = rw_ref[...]
    logits = lax.dot_general(
        x, rw, (((1,), (1,)), ((), ())), preferred_element_type=jnp.float32)
    lane = lax.broadcasted_iota(jnp.int32, logits.shape, 1)
    neg_inf = jnp.float32(-jnp.inf)
    logits = jnp.where(lane < NUM_EXPERTS, logits, neg_inf)
    m1 = jnp.max(logits, axis=1, keepdims=True)
    a1 = jnp.min(jnp.where(logits == m1, lane, LANES), axis=1, keepdims=True)
    masked = jnp.where(lane == a1, neg_inf, logits)
    m2 = jnp.max(masked, axis=1, keepdims=True)
    a2 = jnp.min(jnp.where(masked == m2, lane, LANES), axis=1, keepdims=True)
    p1 = jax.nn.sigmoid(m1 - m2)
    p2 = 1.0 - p1
    ii_ref[...] = jnp.where(lane == 0, a1, jnp.where(lane == 1, a2, 0))
    iw_ref[...] = jnp.where(lane == 0, p1, jnp.where(lane == 1, p2, 0.0))


def _sc_gather_body(x_hbm, idx_hbm, out_hbm, idx_v, buf0_v, buf1_v, sem0,
                    sem1):
    wid = lax.axis_index("s") * 2 + lax.axis_index("c")
    rows_per_w = S_MAX // NW
    base = wid * rows_per_w
    pltpu.sync_copy(idx_hbm.at[pl.ds(base, rows_per_w)], idx_v)
    bufs = (buf0_v, buf1_v)
    sems = (sem0, sem1)
    nch = rows_per_w // GCH
    cps = [None] * nch
    for j in range(nch):
        cps[j] = pltpu.async_copy(
            x_hbm.at[idx_v.at[pl.ds(j * GCH, GCH)]], bufs[j % 2], sems[j % 2])
        if j >= 1:
            cps[j - 1].wait()
            pltpu.sync_copy(bufs[(j - 1) % 2],
                            out_hbm.at[pl.ds(base + (j - 1) * GCH, GCH)])
    cps[nch - 1].wait()
    pltpu.sync_copy(bufs[(nch - 1) % 2],
                    out_hbm.at[pl.ds(base + (nch - 1) * GCH, GCH)])


def _dispatch_body(te_ref, act_ref, xs_ref, w1_ref, w2_ref, w3_ref, rw_ref,
                   out_ref):
    t = pl.program_id(0)

    @pl.when(act_ref[t] == 1)
    def _():
        x = xs_ref[...]
        w1 = w1_ref[0]
        w3 = w3_ref[0]
        w2 = w2_ref[0]
        g = lax.dot_general(
            x, w1, (((1,), (1,)), ((), ())), preferred_element_type=jnp.float32)
        u = lax.dot_general(
            x, w3, (((1,), (1,)), ((), ())), preferred_element_type=jnp.float32)
        h = g * jax.nn.sigmoid(g) * u
        y = lax.dot_general(
            h, w2, (((1,), (1,)), ((), ())), preferred_element_type=jnp.float32)
        out_ref[...] = y * rw_ref[:, 0:1]


def _sc_combine_body(ys_hbm, pos0_hbm, pos1_hbm, out_hbm, i0_v, i1_v,
                     buf0_v, buf1_v, sem):
    wid = lax.axis_index("s") * 2 + lax.axis_index("c")
    rows_per_w = N_TOKENS // NW
    base = wid * rows_per_w
    pltpu.sync_copy(pos0_hbm.at[pl.ds(base, rows_per_w)], i0_v)
    pltpu.sync_copy(pos1_hbm.at[pl.ds(base, rows_per_w)], i1_v)
    c0 = pltpu.async_copy(ys_hbm.at[i0_v], buf0_v, sem)
    c1 = pltpu.async_copy(ys_hbm.at[i1_v], buf1_v, sem)
    c0.wait()
    c1.wait()

    def row(r, carry):
        for c in range(DIM // 16):
            sl = pl.ds(c * 16, 16)
            buf0_v[r, sl] = buf0_v[r, sl] + buf1_v[r, sl]
        return carry

    lax.fori_loop(0, rows_per_w, row, 0)
    pltpu.sync_copy(buf0_v, out_hbm.at[pl.ds(base, rows_per_w)])


def kernel(x, router_w, w1, w2, w3):
    B, T, C = x.shape
    x_flat = x.reshape(-1, C)
    n = x_flat.shape[0]
    rw_pad = jnp.zeros((LANES, C), x.dtype).at[:NUM_EXPERTS].set(router_w)

    ii, iw = pl.pallas_call(
        _router_body,
        out_shape=(
            jax.ShapeDtypeStruct((n, LANES), jnp.int32),
            jax.ShapeDtypeStruct((n, LANES), jnp.float32),
        ),
        in_specs=[
            pl.BlockSpec((n, C), lambda: (0, 0)),
            pl.BlockSpec((LANES, C), lambda: (0, 0)),
        ],
        out_specs=(
            pl.BlockSpec((n, LANES), lambda: (0, 0)),
            pl.BlockSpec((n, LANES), lambda: (0, 0)),
        ),
    )(x_flat, rw_pad)

    e_pair = ii[:, :TOP_K].reshape(-1)
    w_pair = iw[:, :TOP_K].reshape(-1)
    tok = jnp.arange(N_PAIRS, dtype=jnp.int32) // TOP_K
    onehot = (e_pair[:, None] == jnp.arange(NUM_EXPERTS)[None, :])
    onehot = onehot.astype(jnp.int32)
    csum = jnp.cumsum(onehot, axis=0)
    counts = csum[-1]
    rank = jnp.take_along_axis(csum, e_pair[:, None], axis=1)[:, 0] - 1
    padded = ((counts + BT - 1) // BT) * BT
    pstart = jnp.cumsum(padded) - padded
    pos_pair = (pstart[e_pair] + rank).astype(jnp.int32)
    pad_tok = (jnp.arange(S_MAX, dtype=jnp.int32) * 17) % N_TOKENS
    src_tok = pad_tok.at[pos_pair].set(tok)
    row_w = jnp.zeros((S_MAX,), jnp.float32).at[pos_pair].set(w_pair)
    row_w_b = jnp.broadcast_to(row_w[:, None], (S_MAX, 128))
    pos0 = pos_pair[0::TOP_K]
    pos1 = pos_pair[1::TOP_K]
    bound = jnp.cumsum(padded // BT)
    t_ar = jnp.arange(T_MAX, dtype=jnp.int32)
    tile_e = jnp.sum((t_ar[:, None] >= bound[None, :]).astype(jnp.int32), axis=1)
    active = (tile_e < NUM_EXPERTS).astype(jnp.int32)
    tile_e = jnp.minimum(tile_e, NUM_EXPERTS - 1)

    mesh = plsc.VectorSubcoreMesh(
        core_axis_name="c", subcore_axis_name="s", num_cores=2, num_subcores=16)
    xs = pl.kernel(
        _sc_gather_body,
        out_type=jax.ShapeDtypeStruct((S_MAX, C), jnp.float32),
        mesh=mesh,
        scratch_types=[
            pltpu.VMEM((S_MAX // NW,), jnp.int32),
            pltpu.VMEM((GCH, C), jnp.float32),
            pltpu.VMEM((GCH, C), jnp.float32),
            pltpu.SemaphoreType.DMA,
            pltpu.SemaphoreType.DMA,
        ],
    )(x_flat, src_tok)

    grid_spec = pltpu.PrefetchScalarGridSpec(
        num_scalar_prefetch=2,
        grid=(T_MAX,),
        in_specs=[
            pl.BlockSpec((BT, C), lambda t, te, act: (t, 0)),
            pl.BlockSpec((1, HID, C), lambda t, te, act: (te[t], 0, 0)),
            pl.BlockSpec((1, C, HID), lambda t, te, act: (te[t], 0, 0)),
            pl.BlockSpec((1, HID, C), lambda t, te, act: (te[t], 0, 0)),
            pl.BlockSpec((BT, 128), lambda t, te, act: (t, 0)),
        ],
        out_specs=pl.BlockSpec((BT, C), lambda t, te, act: (t, 0)),
    )
    ysw = pl.pallas_call(
        _dispatch_body,
        grid_spec=grid_spec,
        out_shape=jax.ShapeDtypeStruct((S_MAX, C), jnp.float32),
    )(tile_e, active, xs, w1, w2, w3, row_w_b)

    out = pl.kernel(
        _sc_combine_body,
        out_type=jax.ShapeDtypeStruct((n, C), jnp.float32),
        mesh=mesh,
        scratch_types=[
            pltpu.VMEM((N_TOKENS // NW,), jnp.int32),
            pltpu.VMEM((N_TOKENS // NW,), jnp.int32),
            pltpu.VMEM((N_TOKENS // NW, C), jnp.float32),
            pltpu.VMEM((N_TOKENS // NW, C), jnp.float32),
            pltpu.SemaphoreType.DMA,
        ],
    )(ysw, pos0, pos1)
    return out.reshape(B, T, C)


# speedup vs baseline: 1.8460x; 1.1713x over previous
import functools

import jax
import jax.numpy as jnp
from jax import lax
from jax.experimental import pallas as pl
from jax.experimental.pallas import tpu as pltpu
from jax.experimental.pallas import tpu_sc as plsc

DIM = 768
NUM_EXPERTS = 8
TOP_K = 2
HID = int(DIM * 1.5)
N_TOKENS = 2048
N_PAIRS = N_TOKENS * TOP_K
LANES = 128
BT = 256
S_MAX = N_PAIRS + NUM_EXPERTS * BT
T_MAX = S_MAX // BT
NW = 32
RPW = N_TOKENS // NW


def _router_body(x_ref, rw_ref, ii_ref, iw_ref):
    x = x_ref[...]
    rw = rw_ref[...]
    logits = lax.dot_general(
        x, rw, (((1,), (1,)), ((), ())), preferred_element_type=jnp.float32)
    lane = lax.broadcasted_iota(jnp.int32, logits.shape, 1)
    neg_inf = jnp.float32(-jnp.inf)
    logits = jnp.where(lane < NUM_EXPERTS, logits, neg_inf)
    m1 = jnp.max(logits, axis=1, keepdims=True)
    a1 = jnp.min(jnp.where(logits == m1, lane, LANES), axis=1, keepdims=True)
    masked = jnp.where(lane == a1, neg_inf, logits)
    m2 = jnp.max(masked, axis=1, keepdims=True)
    a2 = jnp.min(jnp.where(masked == m2, lane, LANES), axis=1, keepdims=True)
    p1 = jax.nn.sigmoid(m1 - m2)
    p2 = 1.0 - p1
    ii_ref[...] = jnp.where(lane == 0, a1, jnp.where(lane == 1, a2, 0))
    iw_ref[...] = jnp.where(lane == 0, p1, jnp.where(lane == 1, p2, 0.0))


def _sc_scatter_body(x_hbm, pos0_hbm, pos1_hbm, out_hbm, i0_v, i1_v, buf_v,
                     sem0, sem1):
    wid = lax.axis_index("s") * 2 + lax.axis_index("c")
    base = wid * RPW
    pltpu.sync_copy(pos0_hbm.at[wid], i0_v)
    pltpu.sync_copy(pos1_hbm.at[wid], i1_v)
    pltpu.sync_copy(x_hbm.at[pl.ds(base, RPW)], buf_v)
    c0 = pltpu.async_copy(buf_v, out_hbm.at[i0_v], sem0)
    c1 = pltpu.async_copy(buf_v, out_hbm.at[i1_v], sem1)
    c0.wait()
    c1.wait()


def _dispatch_body(te_ref, act_ref, xs_ref, w1_ref, w2_ref, w3_ref, out_ref):
    t = pl.program_id(0)

    @pl.when(act_ref[t] == 1)
    def _():
        x = xs_ref[...]
        w1 = w1_ref[0]
        w3 = w3_ref[0]
        w2 = w2_ref[0]
        g = lax.dot_general(
            x, w1, (((1,), (1,)), ((), ())), preferred_element_type=jnp.float32)
        u = lax.dot_general(
            x, w3, (((1,), (1,)), ((), ())), preferred_element_type=jnp.float32)
        h = g * jax.nn.sigmoid(g) * u
        out_ref[...] = lax.dot_general(
            h, w2, (((1,), (1,)), ((), ())), preferred_element_type=jnp.float32)


def _sc_combine_body(ys_hbm, pos0_hbm, pos1_hbm, w0_hbm, w1_hbm, out_hbm,
                     i0_v, i1_v, w0_v, w1_v, buf0_v, buf1_v, sem):
    wid = lax.axis_index("s") * 2 + lax.axis_index("c")
    base = wid * RPW
    pltpu.sync_copy(pos0_hbm.at[wid], i0_v)
    pltpu.sync_copy(pos1_hbm.at[wid], i1_v)
    pltpu.sync_copy(w0_hbm.at[pl.ds(base, RPW)], w0_v)
    pltpu.sync_copy(w1_hbm.at[pl.ds(base, RPW)], w1_v)
    c0 = pltpu.async_copy(ys_hbm.at[i0_v], buf0_v, sem)
    c1 = pltpu.async_copy(ys_hbm.at[i1_v], buf1_v, sem)
    c0.wait()
    c1.wait()

    def row(r, carry):
        w0b = w0_v[r, :]
        w1b = w1_v[r, :]
        for c in range(DIM // 16):
            sl = pl.ds(c * 16, 16)
            buf0_v[r, sl] = buf0_v[r, sl] * w0b + buf1_v[r, sl] * w1b
        return carry

    lax.fori_loop(0, RPW, row, 0)
    pltpu.sync_copy(buf0_v, out_hbm.at[pl.ds(base, RPW)])


def kernel(x, router_w, w1, w2, w3):
    B, T, C = x.shape
    x_flat = x.reshape(-1, C)
    n = x_flat.shape[0]
    rw_pad = jnp.zeros((LANES, C), x.dtype).at[:NUM_EXPERTS].set(router_w)

    ii, iw = pl.pallas_call(
        _router_body,
        out_shape=(
            jax.ShapeDtypeStruct((n, LANES), jnp.int32),
            jax.ShapeDtypeStruct((n, LANES), jnp.float32),
        ),
        in_specs=[
            pl.BlockSpec((n, C), lambda: (0, 0)),
            pl.BlockSpec((LANES, C), lambda: (0, 0)),
        ],
        out_specs=(
            pl.BlockSpec((n, LANES), lambda: (0, 0)),
            pl.BlockSpec((n, LANES), lambda: (0, 0)),
        ),
    )(x_flat, rw_pad)

    e_pair = ii[:, :TOP_K].reshape(-1)
    onehot = (e_pair[:, None] == jnp.arange(NUM_EXPERTS)[None, :])
    onehot = onehot.astype(jnp.int32)
    csum = jnp.cumsum(onehot, axis=0)
    counts = csum[-1]
    rank = jnp.take_along_axis(csum, e_pair[:, None], axis=1)[:, 0] - 1
    padded = ((counts + BT - 1) // BT) * BT
    pstart = jnp.cumsum(padded) - padded
    pos_pair = (pstart[e_pair] + rank).astype(jnp.int32)
    pos0 = pos_pair[0::TOP_K].reshape(NW, RPW)
    pos1 = pos_pair[1::TOP_K].reshape(NW, RPW)
    wt0 = jnp.broadcast_to(iw[:, 0:1], (n, 16))
    wt1 = jnp.broadcast_to(iw[:, 1:2], (n, 16))
    bound = jnp.cumsum(padded // BT)
    t_ar = jnp.arange(T_MAX, dtype=jnp.int32)
    tile_e = jnp.sum((t_ar[:, None] >= bound[None, :]).astype(jnp.int32), axis=1)
    active = (tile_e < NUM_EXPERTS).astype(jnp.int32)
    tile_e = jnp.minimum(tile_e, NUM_EXPERTS - 1)

    mesh = plsc.VectorSubcoreMesh(
        core_axis_name="c", subcore_axis_name="s", num_cores=2, num_subcores=16)
    xs = pl.kernel(
        _sc_scatter_body,
        out_type=jax.ShapeDtypeStruct((S_MAX, C), jnp.float32),
        mesh=mesh,
        scratch_types=[
            pltpu.VMEM((RPW,), jnp.int32),
            pltpu.VMEM((RPW,), jnp.int32),
            pltpu.VMEM((RPW, C), jnp.float32),
            pltpu.SemaphoreType.DMA,
            pltpu.SemaphoreType.DMA,
        ],
    )(x_flat, pos0, pos1)

    grid_spec = pltpu.PrefetchScalarGridSpec(
        num_scalar_prefetch=2,
        grid=(T_MAX,),
        in_specs=[
            pl.BlockSpec((BT, C), lambda t, te, act: (t, 0)),
            pl.BlockSpec((1, HID, C), lambda t, te, act: (te[t], 0, 0)),
            pl.BlockSpec((1, C, HID), lambda t, te, act: (te[t], 0, 0)),
            pl.BlockSpec((1, HID, C), lambda t, te, act: (te[t], 0, 0)),
        ],
        out_specs=pl.BlockSpec((BT, C), lambda t, te, act: (t, 0)),
    )
    ys = pl.pallas_call(
        _dispatch_body,
        grid_spec=grid_spec,
        out_shape=jax.ShapeDtypeStruct((S_MAX, C), jnp.float32),
    )(tile_e, active, xs, w1, w2, w3)

    out = pl.kernel(
        _sc_combine_body,
        out_type=jax.ShapeDtypeStruct((n, C), jnp.float32),
        mesh=mesh,
        scratch_types=[
            pltpu.VMEM((RPW,), jnp.int32),
            pltpu.VMEM((RPW,), jnp.int32),
            pltpu.VMEM((RPW, 16), jnp.float32),
            pltpu.VMEM((RPW, 16), jnp.float32),
            pltpu.VMEM((RPW, C), jnp.float32),
            pltpu.VMEM((RPW, C), jnp.float32),
            pltpu.SemaphoreType.DMA,
        ],
    )(ys, pos0, pos1, wt0, wt1)
    return out.reshape(B, T, C)


# speedup vs baseline: 2.0600x; 1.1160x over previous
import functools

import jax
import jax.numpy as jnp
from jax import lax
from jax.experimental import pallas as pl
from jax.experimental.pallas import tpu as pltpu
from jax.experimental.pallas import tpu_sc as plsc

DIM = 768
NUM_EXPERTS = 8
TOP_K = 2
HID = int(DIM * 1.5)
N_TOKENS = 2048
N_PAIRS = N_TOKENS * TOP_K
LANES = 128
BT = 256
S_MAX = N_PAIRS + NUM_EXPERTS * BT
T_MAX = S_MAX // BT
NW = 32
RPW = N_TOKENS // NW


def _router_body(x_ref, rw_ref, pos0_ref, pos1_ref, wt0_ref, wt1_ref,
                 bound_ref):
    x = x_ref[...]
    rw = rw_ref[...]
    logits = lax.dot_general(
        x, rw, (((1,), (1,)), ((), ())), preferred_element_type=jnp.float32)
    lane = lax.broadcasted_iota(jnp.int32, logits.shape, 1)
    neg_inf = jnp.float32(-jnp.inf)
    logits = jnp.where(lane < NUM_EXPERTS, logits, neg_inf)
    m1 = jnp.max(logits, axis=1, keepdims=True)
    a1 = jnp.min(jnp.where(logits == m1, lane, LANES), axis=1, keepdims=True)
    masked = jnp.where(lane == a1, neg_inf, logits)
    m2 = jnp.max(masked, axis=1, keepdims=True)
    a2 = jnp.min(jnp.where(masked == m2, lane, LANES), axis=1, keepdims=True)
    p1 = jax.nn.sigmoid(m1 - m2)
    p2 = 1.0 - p1

    oh0 = (lane == a1).astype(jnp.float32)
    oh1 = (lane == a2).astype(jnp.float32)
    s = (oh0 + oh1).astype(jnp.bfloat16)
    n = x.shape[0]
    ri = lax.broadcasted_iota(jnp.int32, (n, n), 0)
    ci = lax.broadcasted_iota(jnp.int32, (n, n), 1)
    ltri = (ri > ci).astype(jnp.bfloat16)
    csum = lax.dot_general(
        ltri, s, (((1,), (0,)), ((), ())), preferred_element_type=jnp.float32)
    counts = csum[n - 1:n, :] + s.astype(jnp.float32)[n - 1:n, :]
    padded = jnp.ceil(counts * (1.0 / BT)) * BT
    li = lax.broadcasted_iota(jnp.int32, (LANES, LANES), 0)
    lj = lax.broadcasted_iota(jnp.int32, (LANES, LANES), 1)
    u_excl = (li < lj).astype(jnp.float32)
    u_incl = (li <= lj).astype(jnp.float32)
    pstart = lax.dot_general(
        padded, u_excl, (((1,), (0,)), ((), ())),
        preferred_element_type=jnp.float32)
    bound = lax.dot_general(
        padded * (1.0 / BT), u_incl, (((1,), (0,)), ((), ())),
        preferred_element_type=jnp.float32)
    slot = pstart + csum
    pos0 = jnp.sum(jnp.where(lane == a1, slot, 0.0), axis=1, keepdims=True)
    pos1 = jnp.sum(jnp.where(lane == a2, slot, 0.0), axis=1, keepdims=True)
    pos0_ref[...] = jnp.broadcast_to(pos0.astype(jnp.int32), pos0_ref.shape)
    pos1_ref[...] = jnp.broadcast_to(pos1.astype(jnp.int32), pos1_ref.shape)
    wt0_ref[...] = jnp.broadcast_to(p1, wt0_ref.shape)
    wt1_ref[...] = jnp.broadcast_to(p2, wt1_ref.shape)
    bound_ref[...] = bound.astype(jnp.int32)


def _sc_scatter_body(x_hbm, pos0_hbm, pos1_hbm, out_hbm, i0_v, i1_v, buf_v,
                     sem0, sem1):
    wid = lax.axis_index("s") * 2 + lax.axis_index("c")
    base = wid * RPW
    pltpu.sync_copy(pos0_hbm.at[wid], i0_v)
    pltpu.sync_copy(pos1_hbm.at[wid], i1_v)
    pltpu.sync_copy(x_hbm.at[pl.ds(base, RPW)], buf_v)
    c0 = pltpu.async_copy(buf_v, out_hbm.at[i0_v], sem0)
    c1 = pltpu.async_copy(buf_v, out_hbm.at[i1_v], sem1)
    c0.wait()
    c1.wait()


def _dispatch_body(te_ref, act_ref, xs_ref, w1_ref, w2_ref, w3_ref, out_ref):
    t = pl.program_id(0)

    @pl.when(act_ref[t] == 1)
    def _():
        x = xs_ref[...]
        w1 = w1_ref[0]
        w3 = w3_ref[0]
        w2 = w2_ref[0]
        g = lax.dot_general(
            x, w1, (((1,), (1,)), ((), ())), preferred_element_type=jnp.float32)
        u = lax.dot_general(
            x, w3, (((1,), (1,)), ((), ())), preferred_element_type=jnp.float32)
        h = g * jax.nn.sigmoid(g) * u
        out_ref[...] = lax.dot_general(
            h, w2, (((1,), (1,)), ((), ())), preferred_element_type=jnp.float32)


def _sc_combine_body(ys_hbm, pos0_hbm, pos1_hbm, w0_hbm, w1_hbm, out_hbm,
                     i0_v, i1_v, w0_v, w1_v, buf0_v, buf1_v, sem):
    wid = lax.axis_index("s") * 2 + lax.axis_index("c")
    base = wid * RPW
    pltpu.sync_copy(pos0_hbm.at[wid], i0_v)
    pltpu.sync_copy(pos1_hbm.at[wid], i1_v)
    pltpu.sync_copy(w0_hbm.at[pl.ds(base, RPW)], w0_v)
    pltpu.sync_copy(w1_hbm.at[pl.ds(base, RPW)], w1_v)
    c0 = pltpu.async_copy(ys_hbm.at[i0_v], buf0_v, sem)
    c1 = pltpu.async_copy(ys_hbm.at[i1_v], buf1_v, sem)
    c0.wait()
    c1.wait()

    def row(r, carry):
        w0b = w0_v[r, :]
        w1b = w1_v[r, :]
        for c in range(DIM // 16):
            sl = pl.ds(c * 16, 16)
            buf0_v[r, sl] = buf0_v[r, sl] * w0b + buf1_v[r, sl] * w1b
        return carry

    lax.fori_loop(0, RPW, row, 0)
    pltpu.sync_copy(buf0_v, out_hbm.at[pl.ds(base, RPW)])


def kernel(x, router_w, w1, w2, w3):
    B, T, C = x.shape
    x_flat = x.reshape(-1, C)
    n = x_flat.shape[0]
    rw_pad = jnp.zeros((LANES, C), x.dtype).at[:NUM_EXPERTS].set(router_w)

    pos0_w, pos1_w, wt0, wt1, bound_w = pl.pallas_call(
        _router_body,
        out_shape=(
            jax.ShapeDtypeStruct((n, 16), jnp.int32),
            jax.ShapeDtypeStruct((n, 16), jnp.int32),
            jax.ShapeDtypeStruct((n, 16), jnp.float32),
            jax.ShapeDtypeStruct((n, 16), jnp.float32),
            jax.ShapeDtypeStruct((1, LANES), jnp.int32),
        ),
        in_specs=[
            pl.BlockSpec((n, C), lambda: (0, 0)),
            pl.BlockSpec((LANES, C), lambda: (0, 0)),
        ],
        out_specs=(
            pl.BlockSpec((n, 16), lambda: (0, 0)),
            pl.BlockSpec((n, 16), lambda: (0, 0)),
            pl.BlockSpec((n, 16), lambda: (0, 0)),
            pl.BlockSpec((n, 16), lambda: (0, 0)),
            pl.BlockSpec((1, LANES), lambda: (0, 0)),
        ),
    )(x_flat, rw_pad)

    pos0 = pos0_w[:, 0].reshape(NW, RPW)
    pos1 = pos1_w[:, 0].reshape(NW, RPW)
    bound = bound_w[0, :NUM_EXPERTS]
    t_ar = jnp.arange(T_MAX, dtype=jnp.int32)
    tile_e = jnp.sum((t_ar[:, None] >= bound[None, :]).astype(jnp.int32), axis=1)
    active = (tile_e < NUM_EXPERTS).astype(jnp.int32)
    tile_e = jnp.minimum(tile_e, NUM_EXPERTS - 1)

    mesh = plsc.VectorSubcoreMesh(
        core_axis_name="c", subcore_axis_name="s", num_cores=2, num_subcores=16)
    xs = pl.kernel(
        _sc_scatter_body,
        out_type=jax.ShapeDtypeStruct((S_MAX, C), jnp.float32),
        mesh=mesh,
        scratch_types=[
            pltpu.VMEM((RPW,), jnp.int32),
            pltpu.VMEM((RPW,), jnp.int32),
            pltpu.VMEM((RPW, C), jnp.float32),
            pltpu.SemaphoreType.DMA,
            pltpu.SemaphoreType.DMA,
        ],
    )(x_flat, pos0, pos1)

    grid_spec = pltpu.PrefetchScalarGridSpec(
        num_scalar_prefetch=2,
        grid=(T_MAX,),
        in_specs=[
            pl.BlockSpec((BT, C), lambda t, te, act: (t, 0)),
            pl.BlockSpec((1, HID, C), lambda t, te, act: (te[t], 0, 0)),
            pl.BlockSpec((1, C, HID), lambda t, te, act: (te[t], 0, 0)),
            pl.BlockSpec((1, HID, C), lambda t, te, act: (te[t], 0, 0)),
        ],
        out_specs=pl.BlockSpec((BT, C), lambda t, te, act: (t, 0)),
    )
    ys = pl.pallas_call(
        _dispatch_body,
        grid_spec=grid_spec,
        out_shape=jax.ShapeDtypeStruct((S_MAX, C), jnp.float32),
    )(tile_e, active, xs, w1, w2, w3)

    out = pl.kernel(
        _sc_combine_body,
        out_type=jax.ShapeDtypeStruct((n, C), jnp.float32),
        mesh=mesh,
        scratch_types=[
            pltpu.VMEM((RPW,), jnp.int32),
            pltpu.VMEM((RPW,), jnp.int32),
            pltpu.VMEM((RPW, 16), jnp.float32),
            pltpu.VMEM((RPW, 16), jnp.float32),
            pltpu.VMEM((RPW, C), jnp.float32),
            pltpu.VMEM((RPW, C), jnp.float32),
            pltpu.SemaphoreType.DMA,
        ],
    )(ys, pos0, pos1, wt0, wt1)
    return out.reshape(B, T, C)


# speedup vs baseline: 2.3862x; 1.1583x over previous
import jax
import jax.numpy as jnp
from jax import lax
from jax.experimental import pallas as pl
from jax.experimental.pallas import tpu as pltpu
from jax.experimental.pallas import tpu_sc as plsc

DIM = 768
NUM_EXPERTS = 8
TOP_K = 2
HID = int(DIM * 1.5)
N_TOKENS = 2048
N_PAIRS = N_TOKENS * TOP_K
LANES = 128
BT = 256
S_MAX = N_PAIRS + NUM_EXPERTS * BT
T_MAX = S_MAX // BT
NW = 32
RPW = N_TOKENS // NW


def _router_body(x_ref, rw_ref, pos0_ref, pos1_ref, wt0_ref, wt1_ref,
                 bound_ref):
    x = x_ref[...]
    rw = rw_ref[...]
    logits = lax.dot_general(
        x, rw, (((1,), (1,)), ((), ())), preferred_element_type=jnp.float32)
    lane = lax.broadcasted_iota(jnp.int32, logits.shape, 1)
    neg_inf = jnp.float32(-jnp.inf)
    logits = jnp.where(lane < NUM_EXPERTS, logits, neg_inf)
    m1 = jnp.max(logits, axis=1, keepdims=True)
    a1 = jnp.min(jnp.where(logits == m1, lane, LANES), axis=1, keepdims=True)
    masked = jnp.where(lane == a1, neg_inf, logits)
    m2 = jnp.max(masked, axis=1, keepdims=True)
    a2 = jnp.min(jnp.where(masked == m2, lane, LANES), axis=1, keepdims=True)
    p1 = jax.nn.sigmoid(m1 - m2)
    p2 = 1.0 - p1

    oh0 = (lane == a1).astype(jnp.float32)
    oh1 = (lane == a2).astype(jnp.float32)
    s = (oh0 + oh1).astype(jnp.bfloat16)
    n = x.shape[0]
    ri = lax.broadcasted_iota(jnp.int32, (n, n), 0)
    ci = lax.broadcasted_iota(jnp.int32, (n, n), 1)
    ltri = (ri > ci).astype(jnp.bfloat16)
    csum = lax.dot_general(
        ltri, s, (((1,), (0,)), ((), ())), preferred_element_type=jnp.float32)
    counts = csum[n - 1:n, :] + s.astype(jnp.float32)[n - 1:n, :]
    padded = jnp.ceil(counts * (1.0 / BT)) * BT
    li = lax.broadcasted_iota(jnp.int32, (LANES, LANES), 0)
    lj = lax.broadcasted_iota(jnp.int32, (LANES, LANES), 1)
    u_excl = (li < lj).astype(jnp.float32)
    u_incl = (li <= lj).astype(jnp.float32)
    pstart = lax.dot_general(
        padded, u_excl, (((1,), (0,)), ((), ())),
        preferred_element_type=jnp.float32)
    bound = lax.dot_general(
        padded * (1.0 / BT), u_incl, (((1,), (0,)), ((), ())),
        preferred_element_type=jnp.float32)
    slot = pstart + csum
    pos0 = jnp.sum(jnp.where(lane == a1, slot, 0.0), axis=1, keepdims=True)
    pos1 = jnp.sum(jnp.where(lane == a2, slot, 0.0), axis=1, keepdims=True)
    pos0_ref[...] = jnp.broadcast_to(pos0.astype(jnp.int32), pos0_ref.shape)
    pos1_ref[...] = jnp.broadcast_to(pos1.astype(jnp.int32), pos1_ref.shape)
    wt0_ref[...] = jnp.broadcast_to(p1, wt0_ref.shape)
    wt1_ref[...] = jnp.broadcast_to(p2, wt1_ref.shape)
    bound_ref[...] = bound.astype(jnp.int32)


def _sc_scatter_body(x_hbm, pos0_hbm, pos1_hbm, out_hbm, i0_v, i1_v, buf_v,
                     sem0, sem1):
    wid = lax.axis_index("s") * 2 + lax.axis_index("c")
    base = wid * RPW
    pltpu.sync_copy(pos0_hbm.at[wid], i0_v)
    pltpu.sync_copy(pos1_hbm.at[wid], i1_v)
    pltpu.sync_copy(x_hbm.at[pl.ds(base, RPW)], buf_v)
    c0 = pltpu.async_copy(buf_v, out_hbm.at[i0_v], sem0)
    c1 = pltpu.async_copy(buf_v, out_hbm.at[i1_v], sem1)
    c0.wait()
    c1.wait()


def _dispatch_body(te_ref, act_ref, first_ref, nxt_ref, slot_ref, mapix_ref,
                   xs_ref, w1_any, w2_any, w3_any, out_ref, w1b, w2b, w3b,
                   sems):
    t = pl.program_id(0)
    slot = slot_ref[t]

    def issue(e, s):
        pltpu.make_async_copy(w1_any.at[e], w1b.at[s], sems.at[s]).start()
        pltpu.make_async_copy(w3_any.at[e], w3b.at[s], sems.at[s]).start()
        pltpu.make_async_copy(w2_any.at[e], w2b.at[s], sems.at[s]).start()

    @pl.when(t == 0)
    def _():
        issue(te_ref[0], slot)

    @pl.when((first_ref[t] == 1) & (nxt_ref[t] >= 0))
    def _():
        issue(nxt_ref[t], 1 - slot)

    @pl.when(first_ref[t] == 1)
    def _():
        pltpu.make_async_copy(w1_any.at[0], w1b.at[slot], sems.at[slot]).wait()
        pltpu.make_async_copy(w3_any.at[0], w3b.at[slot], sems.at[slot]).wait()
        pltpu.make_async_copy(w2_any.at[0], w2b.at[slot], sems.at[slot]).wait()

    @pl.when(act_ref[t] == 1)
    def _():
        x = xs_ref[...]
        w1 = w1b[slot]
        w3 = w3b[slot]
        w2 = w2b[slot]
        g = lax.dot_general(
            x, w1, (((1,), (1,)), ((), ())), preferred_element_type=jnp.float32)
        u = lax.dot_general(
            x, w3, (((1,), (1,)), ((), ())), preferred_element_type=jnp.float32)
        h = g * jax.nn.sigmoid(g) * u
        out_ref[...] = lax.dot_general(
            h, w2, (((1,), (1,)), ((), ())), preferred_element_type=jnp.float32)


def _sc_combine_body(ys_hbm, pos0_hbm, pos1_hbm, w0_hbm, w1_hbm, out_hbm,
                     i0_v, i1_v, w0_v, w1_v, buf0_v, buf1_v, sem0, sem1):
    wid = lax.axis_index("s") * 2 + lax.axis_index("c")
    base = wid * RPW
    pltpu.sync_copy(pos0_hbm.at[wid], i0_v)
    pltpu.sync_copy(pos1_hbm.at[wid], i1_v)
    pltpu.sync_copy(w0_hbm.at[pl.ds(base, RPW)], w0_v)
    pltpu.sync_copy(w1_hbm.at[pl.ds(base, RPW)], w1_v)
    c0 = pltpu.async_copy(ys_hbm.at[i0_v], buf0_v, sem0)
    c1 = pltpu.async_copy(ys_hbm.at[i1_v], buf1_v, sem1)
    c0.wait()
    c1.wait()

    def row(r, carry):
        w0b = w0_v[r, :]
        w1b = w1_v[r, :]
        for c in range(DIM // 16):
            sl = pl.ds(c * 16, 16)
            buf0_v[r, sl] = buf0_v[r, sl] * w0b + buf1_v[r, sl] * w1b
        return carry

    lax.fori_loop(0, RPW, row, 0)
    pltpu.sync_copy(buf0_v, out_hbm.at[pl.ds(base, RPW)])


def kernel(x, router_w, w1, w2, w3):
    B, T, C = x.shape
    x_flat = x.reshape(-1, C)
    n = x_flat.shape[0]
    rw_pad = jnp.zeros((LANES, C), x.dtype).at[:NUM_EXPERTS].set(router_w)

    pos0_w, pos1_w, wt0, wt1, bound_w = pl.pallas_call(
        _router_body,
        out_shape=(
            jax.ShapeDtypeStruct((n, 16), jnp.int32),
            jax.ShapeDtypeStruct((n, 16), jnp.int32),
            jax.ShapeDtypeStruct((n, 16), jnp.float32),
            jax.ShapeDtypeStruct((n, 16), jnp.float32),
            jax.ShapeDtypeStruct((1, LANES), jnp.int32),
        ),
        in_specs=[
            pl.BlockSpec((n, C), lambda: (0, 0)),
            pl.BlockSpec((LANES, C), lambda: (0, 0)),
        ],
        out_specs=(
            pl.BlockSpec((n, 16), lambda: (0, 0)),
            pl.BlockSpec((n, 16), lambda: (0, 0)),
            pl.BlockSpec((n, 16), lambda: (0, 0)),
            pl.BlockSpec((n, 16), lambda: (0, 0)),
            pl.BlockSpec((1, LANES), lambda: (0, 0)),
        ),
    )(x_flat, rw_pad)

    pos0 = pos0_w[:, 0].reshape(NW, RPW)
    pos1 = pos1_w[:, 0].reshape(NW, RPW)
    bound = bound_w[0, :NUM_EXPERTS]
    t_ar = jnp.arange(T_MAX, dtype=jnp.int32)
    tile_e = jnp.sum((t_ar[:, None] >= bound[None, :]).astype(jnp.int32), axis=1)
    active = (tile_e < NUM_EXPERTS).astype(jnp.int32)
    tile_e = jnp.minimum(tile_e, NUM_EXPERTS - 1)
    prev_e = jnp.concatenate([jnp.full((1,), -1, jnp.int32), tile_e[:-1]])
    first = (active == 1) & (tile_e != prev_e)
    slot = (jnp.cumsum(first.astype(jnp.int32)) - 1) % 2
    ntiles = bound - jnp.concatenate([jnp.zeros((1,), bound.dtype), bound[:-1]])
    e_ar = jnp.arange(NUM_EXPERTS, dtype=jnp.int32)
    cand = jnp.where((e_ar[None, :] > e_ar[:, None]) & (ntiles[None, :] > 0),
                     e_ar[None, :], NUM_EXPERTS)
    next_active = jnp.min(cand, axis=1)
    next_active = jnp.where(next_active < NUM_EXPERTS, next_active, -1)
    nxt = jnp.where(first, next_active[tile_e], -1).astype(jnp.int32)
    first = first.astype(jnp.int32)
    slot = slot.astype(jnp.int32)
    n_act = jnp.sum(active)
    mapix = jnp.minimum(t_ar, jnp.maximum(n_act - 1, 0)).astype(jnp.int32)

    mesh = plsc.VectorSubcoreMesh(
        core_axis_name="c", subcore_axis_name="s", num_cores=2, num_subcores=16)
    xs = pl.kernel(
        _sc_scatter_body,
        out_type=jax.ShapeDtypeStruct((S_MAX, C), jnp.float32),
        mesh=mesh,
        scratch_types=[
            pltpu.VMEM((RPW,), jnp.int32),
            pltpu.VMEM((RPW,), jnp.int32),
            pltpu.VMEM((RPW, C), jnp.float32),
            pltpu.SemaphoreType.DMA,
            pltpu.SemaphoreType.DMA,
        ],
    )(x_flat, pos0, pos1)

    grid_spec = pltpu.PrefetchScalarGridSpec(
        num_scalar_prefetch=6,
        grid=(T_MAX,),
        in_specs=[
            pl.BlockSpec((BT, C), lambda t, *p: (p[5][t], 0)),
            pl.BlockSpec(memory_space=pltpu.MemorySpace.HBM),
            pl.BlockSpec(memory_space=pltpu.MemorySpace.HBM),
            pl.BlockSpec(memory_space=pltpu.MemorySpace.HBM),
        ],
        out_specs=pl.BlockSpec((BT, C), lambda t, *p: (p[5][t], 0)),
        scratch_shapes=[
            pltpu.VMEM((2, HID, C), jnp.float32),
            pltpu.VMEM((2, C, HID), jnp.float32),
            pltpu.VMEM((2, HID, C), jnp.float32),
            pltpu.SemaphoreType.DMA((2,)),
        ],
    )
    ys = pl.pallas_call(
        _dispatch_body,
        grid_spec=grid_spec,
        out_shape=jax.ShapeDtypeStruct((S_MAX, C), jnp.float32),
    )(tile_e, active, first, nxt, slot, mapix, xs, w1, w2, w3)

    out = pl.kernel(
        _sc_combine_body,
        out_type=jax.ShapeDtypeStruct((n, C), jnp.float32),
        mesh=mesh,
        scratch_types=[
            pltpu.VMEM((RPW,), jnp.int32),
            pltpu.VMEM((RPW,), jnp.int32),
            pltpu.VMEM((RPW, 16), jnp.float32),
            pltpu.VMEM((RPW, 16), jnp.float32),
            pltpu.VMEM((RPW, C), jnp.float32),
            pltpu.VMEM((RPW, C), jnp.float32),
            pltpu.SemaphoreType.DMA,
            pltpu.SemaphoreType.DMA,
        ],
    )(ys, pos0, pos1, wt0, wt1)
    return out.reshape(B, T, C)
